# baseline (speedup 1.0000x reference)
import functools

import jax
import jax.numpy as jnp
from jax import lax
from jax.experimental import pallas as pl
from jax.experimental.pallas import tpu as pltpu
from jax.experimental.pallas import tpu_sc as plsc

NC = 22
KQ = 1024
H = 1024
E = 1024
CDIM = 128
DR = 2048
DF = 2048
B = 16
T = 32

_QROWS = NC * CDIM
_NSUB = 16
_NW = 32
_WROWS = _QROWS // _NW
_PROWS = 8
_NP = _WROWS // _PROWS


_DN = (((1,), (1,)), ((), ()))


def _enc_body(rgba_ref, flowa_ref, rgbs_ref, flows_ref, w1_ref,
              b1_ref, g1_ref, be1_ref, wih_ref, gi_ref, g0_ref):
    xa = jnp.concatenate(
        [rgba_ref[...].astype(jnp.bfloat16).reshape(B * T, DR),
         flowa_ref[...].astype(jnp.bfloat16).reshape(B * T, DF)], axis=1)
    xs = jnp.concatenate(
        [rgbs_ref[...].astype(jnp.bfloat16).reshape(B * T, DR),
         flows_ref[...].astype(jnp.bfloat16).reshape(B * T, DF)], axis=1)
    x = jnp.concatenate([xa, xs], axis=0)
    p = lax.dot_general(x, w1_ref[...], _DN,
                        preferred_element_type=jnp.float32)
    p = p + b1_ref[...]
    mu = jnp.mean(p, axis=1, keepdims=True)
    var = jnp.mean((p - mu) ** 2, axis=1, keepdims=True)
    y = (p - mu) / jnp.sqrt(var + 1e-5) * g1_ref[...] + be1_ref[...]
    y = jnp.maximum(y, 0.0).astype(jnp.bfloat16)
    g = lax.dot_general(y, wih_ref[...], _DN,
                        preferred_element_type=jnp.float32)
    g = jnp.transpose(g.reshape(2, B, T, 3 * H), (2, 0, 1, 3))
    gi_ref[...] = g.astype(jnp.bfloat16)
    b1v = b1_ref[...]
    mu0 = jnp.mean(b1v, axis=1, keepdims=True)
    var0 = jnp.mean((b1v - mu0) ** 2, axis=1, keepdims=True)
    y0 = (b1v - mu0) / jnp.sqrt(var0 + 1e-5) * g1_ref[...] + be1_ref[...]
    y0 = jnp.maximum(y0, 0.0).astype(jnp.bfloat16)
    g0_ref[...] = lax.dot_general(y0, wih_ref[...], _DN,
                                  preferred_element_type=jnp.float32)


def _encode(rgb_a, flow_a, rgb_s, flow_s, w1, b1, g1, be1, wih,
            interpret=False):
    return pl.pallas_call(
        _enc_body,
        out_shape=(jax.ShapeDtypeStruct((T, 2, B, 3 * H), jnp.bfloat16),
                   jax.ShapeDtypeStruct((1, 3 * H), jnp.float32)),
        interpret=interpret,
    )(rgb_a, flow_a, rgb_s, flow_s, w1, b1, g1, be1, wih)


def _gru_body(gi_ref, g0_ref, mc_ref, mx_ref, whh_ref, bih_ref, bhh_ref,
              wh1_ref, bh1_ref, wh2_ref, bh2_ref, q_ref, h_ref):
    h_ref[...] = jnp.zeros((4 * B, H), jnp.float32)

    def step(t, carry):
        gk = gi_ref[t, 0].astype(jnp.float32)
        gs = gi_ref[t, 1].astype(jnp.float32)
        g0 = g0_ref[...]
        core = jnp.where(mc_ref[t] > 0.0, gk, g0)
        ctx = jnp.where(mx_ref[t] > 0.0, gk, g0)
        gi = jnp.concatenate([core, ctx, gk, gs], axis=0) + bih_ref[...]
        h = h_ref[...]
        gh = lax.dot_general(h.astype(jnp.bfloat16), whh_ref[...], _DN,
                             preferred_element_type=jnp.float32) + bhh_ref[...]
        r = jax.nn.sigmoid(gi[:, :H] + gh[:, :H])
        z = jax.nn.sigmoid(gi[:, H:2 * H] + gh[:, H:2 * H])
        n = jnp.tanh(gi[:, 2 * H:] + r * gh[:, 2 * H:])
        h_ref[...] = (1.0 - z) * n + z * h
        return carry

    lax.fori_loop(0, T, step, 0)
    h = jnp.maximum(h_ref[...], 0.0)
    t1 = lax.dot_general(h.astype(jnp.bfloat16), wh1_ref[...], _DN,
                         preferred_element_type=jnp.float32) + bh1_ref[...]
    t1 = jnp.maximum(t1, 0.0)
    q = lax.dot_general(t1.astype(jnp.bfloat16), wh2_ref[...], _DN,
                        preferred_element_type=jnp.float32) + bh2_ref[...]
    nrm = jnp.sqrt(jnp.sum(q * q, axis=1, keepdims=True))
    q = q / jnp.maximum(nrm, 1e-12)
    q_ref[...] = q.reshape(4, B, CDIM)


def _gru_head(gi, g0, mc, mx, whh_t, bih, bhh, wh1_t, bh1, wh2_t, bh2,
              interpret=False):
    return pl.pallas_call(
        _gru_body,
        out_shape=jax.ShapeDtypeStruct((4, B, CDIM), jnp.float32),
        scratch_shapes=[pltpu.VMEM((4 * B, H), jnp.float32)],
        interpret=interpret,
    )(gi, g0, mc, mx, whh_t, bih, bhh, wh1_t, bh1, wh2_t, bh2)


def _queue_body(q_hbm, k_hbm, lbl_hbm, ptr_hbm, outq_hbm, outp_hbm,
                lbl_v, ptr_v, k_v, np_v, buf_a, buf_b,
                sem_ra, sem_rb, sem_wa, sem_wb):
    cid = lax.axis_index("c")
    sid = lax.axis_index("s")
    wid = cid * _NSUB + sid

    pltpu.sync_copy(lbl_hbm, lbl_v)
    pltpu.sync_copy(ptr_hbm, ptr_v)
    pltpu.sync_copy(k_hbm, k_v)
    iot = lax.iota(jnp.int32, 16)
    lblv = lbl_v[...]
    offs = jnp.zeros((16,), jnp.int32)
    for j in range(16):
        lj = jnp.full((16,), jnp.sum(jnp.where(iot == j, lblv, 0)))
        offs = offs + ((lj == lblv) & (iot > j)).astype(jnp.int32)
    ptrg = plsc.load_gather(ptr_v, [lblv])
    posv = lax.rem(ptrg + offs, jnp.full((16,), KQ, jnp.int32))
    zt = jnp.sum(jnp.where(iot == 0, lblv, 0)) * 0

    base = wid * _WROWS
    bufs = (buf_a, buf_b)
    rsems = (sem_ra, sem_rb)
    wsems = (sem_wa, sem_wb)

    def patch(p, buf):
        r0 = base + p * _PROWS
        cls_p = lax.div(r0, CDIM)
        off = lax.rem(r0, CDIM)
        m = lblv == cls_p
        for j in range(_PROWS):
            col = jnp.full((16,), off + j)
            vals = plsc.load_gather(k_v, [iot, col])
            rows = jnp.full((16,), j + zt)
            plsc.store_scatter(buf, [rows, posv], vals, mask=m)

    rd = pltpu.async_copy(q_hbm.at[pl.ds(base, _PROWS)], bufs[0], rsems[0])
    wr = None
    for p in range(_NP):
        rd.wait()
        if wr is not None:
            wr.wait()
        if p + 1 < _NP:
            rd = pltpu.async_copy(
                q_hbm.at[pl.ds(base + (p + 1) * _PROWS, _PROWS)],
                bufs[(p + 1) % 2], rsems[(p + 1) % 2])
        patch(p, bufs[p % 2])
        wr = pltpu.async_copy(bufs[p % 2],
                              outq_hbm.at[pl.ds(base + p * _PROWS, _PROWS)],
                              wsems[p % 2])
    wr.wait()

    @pl.when((cid == 0) & (sid == 0))
    def _():
        for half in range(2):
            clsv = half * 16 + iot
            cnt = jnp.zeros((16,), jnp.int32)
            for i in range(16):
                liv = jnp.full((16,), jnp.sum(jnp.where(iot == i, lblv, 0)))
                cnt = cnt + (liv == clsv).astype(jnp.int32)
            np_v[pl.ds(half * 16, 16)] = lax.rem(
                ptr_v[pl.ds(half * 16, 16)] + cnt,
                jnp.full((16,), KQ, jnp.int32))
        pltpu.sync_copy(np_v, outp_hbm)


def _queue_update(q2d, k_cls, lbl16, ptr32, interpret=False):
    mesh = plsc.VectorSubcoreMesh(core_axis_name="c", subcore_axis_name="s",
                                  num_cores=2, num_subcores=_NSUB)
    f = functools.partial(
        pl.kernel,
        out_type=(jax.ShapeDtypeStruct((_QROWS, KQ), jnp.float32),
                  jax.ShapeDtypeStruct((32,), jnp.int32)),
        mesh=mesh,
        scratch_types=[
            pltpu.VMEM((16,), jnp.int32),
            pltpu.VMEM((32,), jnp.int32),
            pltpu.VMEM((16, CDIM), jnp.float32),
            pltpu.VMEM((32,), jnp.int32),
            pltpu.VMEM((_PROWS, KQ), jnp.float32),
            pltpu.VMEM((_PROWS, KQ), jnp.float32),
            pltpu.SemaphoreType.DMA,
            pltpu.SemaphoreType.DMA,
            pltpu.SemaphoreType.DMA,
            pltpu.SemaphoreType.DMA,
        ],
        compiler_params=pltpu.CompilerParams(needs_layout_passes=False),
        interpret=interpret,
    )(_queue_body)
    return f(q2d, k_cls, lbl16, ptr32)


def kernel(rgb_anchor, flow_anchor, rgb_shuff, flow_shuff, labels,
           labels_per_frame, W1, b1, g1, be1, W_ih, W_hh, b_ih, b_hh,
           Wh1, bh1, Wh2, bh2, queues, queue_ptrs):
    rk = jax.random.key(42)
    rand = jax.random.uniform(rk, (B, T - 1, 1))
    mask_random = jnp.concatenate(
        [(rand > 0.0).astype(jnp.float32), jnp.ones((B, 1, 1), jnp.float32)],
        axis=1)
    is_bg = (labels_per_frame == 0)[..., None].astype(jnp.float32)
    mask_core_sem = 1.0 - is_bg
    is_bg_sample = (labels == 0).reshape(B, 1, 1)
    mask_core = jnp.where(is_bg_sample, mask_random, mask_core_sem)
    has_action = jnp.sum(mask_core, axis=1, keepdims=True) > 0
    mask_core = jnp.where(has_action, mask_core, mask_random)
    mask_ctx = jnp.where(is_bg_sample, jnp.zeros_like(is_bg), is_bg)
    mc_t = jnp.transpose(mask_core, (1, 0, 2))
    mx_t = jnp.transpose(mask_ctx, (1, 0, 2))

    bf = jnp.bfloat16
    w1b = W1.astype(bf)
    wihb = W_ih.astype(bf)
    whhb = W_hh.astype(bf)
    wh1b = Wh1.astype(bf)
    wh2b = Wh2.astype(bf)
    b1r = b1.reshape(1, E)
    g1r = g1.reshape(1, E)
    be1r = be1.reshape(1, E)
    bihr = b_ih.reshape(1, 3 * H)
    bhhr = b_hh.reshape(1, 3 * H)
    bh1r = bh1.reshape(1, H)
    bh2r = bh2.reshape(1, CDIM)

    gi, g0 = _encode(rgb_anchor, flow_anchor, rgb_shuff, flow_shuff,
                     w1b, b1r, g1r, be1r, wihb)

    q = _gru_head(gi, g0, mc_t, mx_t, whhb, bihr, bhhr, wh1b, bh1r,
                  wh2b, bh2r)
    q_cls, q_ctx, k_cls, q_shf = q[0], q[1], q[2], q[3]

    lbl16 = labels.astype(jnp.int32)
    ptr32 = jnp.pad(queue_ptrs.astype(jnp.int32), (0, 32 - NC))
    outq, outp = _queue_update(queues.reshape(_QROWS, KQ), k_cls, lbl16, ptr32)
    new_queues = outq.reshape(NC, CDIM, KQ)
    new_ptrs = outp[:NC]

    return (q_cls, k_cls, q_shf, q_ctx, new_queues, new_ptrs)


# speedup vs baseline: 1.0763x; 1.0763x over previous
import functools

import jax
import jax.numpy as jnp
from jax import lax
from jax.experimental import pallas as pl
from jax.experimental.pallas import tpu as pltpu
from jax.experimental.pallas import tpu_sc as plsc

NC = 22
KQ = 1024
H = 1024
E = 1024
CDIM = 128
DR = 2048
DF = 2048
B = 16
T = 32

_QROWS = NC * CDIM
_NSUB = 16
_NW = 32
_WROWS = _QROWS // _NW
_PROWS = 8
_NP = _WROWS // _PROWS


def _enc_body(rgba_ref, flowa_ref, rgbs_ref, flows_ref, w1r_ref, w1f_ref,
              b1_ref, g1_ref, be1_ref, wih_ref, gi_ref, g0_ref):
    xr = jnp.concatenate(
        [rgba_ref[...].astype(jnp.bfloat16).reshape(B * T, DR),
         rgbs_ref[...].astype(jnp.bfloat16).reshape(B * T, DR)], axis=0)
    xf = jnp.concatenate(
        [flowa_ref[...].astype(jnp.bfloat16).reshape(B * T, DF),
         flows_ref[...].astype(jnp.bfloat16).reshape(B * T, DF)], axis=0)
    p = jnp.dot(xr, w1r_ref[...], preferred_element_type=jnp.float32)
    p = p + jnp.dot(xf, w1f_ref[...], preferred_element_type=jnp.float32)
    p = p + b1_ref[...]
    mu = jnp.mean(p, axis=1, keepdims=True)
    var = jnp.mean((p - mu) ** 2, axis=1, keepdims=True)
    y = (p - mu) / jnp.sqrt(var + 1e-5) * g1_ref[...] + be1_ref[...]
    y = jnp.maximum(y, 0.0).astype(jnp.bfloat16)
    g = jnp.dot(y, wih_ref[...], preferred_element_type=jnp.float32)
    g = jnp.transpose(g.reshape(2, B, T, 3 * H), (2, 0, 1, 3))
    gi_ref[...] = g.astype(jnp.bfloat16)
    b1v = b1_ref[...]
    mu0 = jnp.mean(b1v, axis=1, keepdims=True)
    var0 = jnp.mean((b1v - mu0) ** 2, axis=1, keepdims=True)
    y0 = (b1v - mu0) / jnp.sqrt(var0 + 1e-5) * g1_ref[...] + be1_ref[...]
    y0 = jnp.maximum(y0, 0.0).astype(jnp.bfloat16)
    g0_ref[...] = jnp.dot(y0, wih_ref[...], preferred_element_type=jnp.float32)


def _encode(rgb_a, flow_a, rgb_s, flow_s, w1r_t, w1f_t, b1, g1, be1, wih_t,
            interpret=False):
    return pl.pallas_call(
        _enc_body,
        out_shape=(jax.ShapeDtypeStruct((T, 2, B, 3 * H), jnp.bfloat16),
                   jax.ShapeDtypeStruct((1, 3 * H), jnp.float32)),
        interpret=interpret,
    )(rgb_a, flow_a, rgb_s, flow_s, w1r_t, w1f_t, b1, g1, be1, wih_t)


def _fused_body(rgba_ref, flowa_ref, rgbs_ref, flows_ref, mc_ref, mx_ref,
                w1r_ref, w1f_ref, b1_ref, g1_ref, be1_ref, wih_ref,
                whh_ref, bih_ref, bhh_ref, wh1_ref, bh1_ref, wh2_ref,
                bh2_ref, q_ref, gi_s, g0_s, h_ref):
    _enc_body(rgba_ref, flowa_ref, rgbs_ref, flows_ref, w1r_ref, w1f_ref,
              b1_ref, g1_ref, be1_ref, wih_ref, gi_s, g0_s)
    _gru_body(gi_s, g0_s, mc_ref, mx_ref, whh_ref, bih_ref, bhh_ref,
              wh1_ref, bh1_ref, wh2_ref, bh2_ref, q_ref, h_ref)


def _fused_call(rgb_a, flow_a, rgb_s, flow_s, mc, mx, w1r_t, w1f_t, b1, g1,
                be1, wih_t, whh_t, bih, bhh, wh1_t, bh1, wh2_t, bh2,
                interpret=False):
    return pl.pallas_call(
        _fused_body,
        out_shape=jax.ShapeDtypeStruct((4, B, CDIM), jnp.float32),
        scratch_shapes=[pltpu.VMEM((T, 2, B, 3 * H), jnp.bfloat16),
                        pltpu.VMEM((1, 3 * H), jnp.float32),
                        pltpu.VMEM((4 * B, H), jnp.float32)],
        interpret=interpret,
    )(rgb_a, flow_a, rgb_s, flow_s, mc, mx, w1r_t, w1f_t, b1, g1, be1,
      wih_t, whh_t, bih, bhh, wh1_t, bh1, wh2_t, bh2)


def _gru_body(gi_ref, g0_ref, mc_ref, mx_ref, whh_ref, bih_ref, bhh_ref,
              wh1_ref, bh1_ref, wh2_ref, bh2_ref, q_ref, h_ref):
    h_ref[...] = jnp.zeros((4 * B, H), jnp.float32)

    def step(t, carry):
        gk = gi_ref[t, 0].astype(jnp.float32)
        gs = gi_ref[t, 1].astype(jnp.float32)
        g0 = g0_ref[...]
        core = jnp.where(mc_ref[t] > 0.0, gk, g0)
        ctx = jnp.where(mx_ref[t] > 0.0, gk, g0)
        gi = jnp.concatenate([core, ctx, gk, gs], axis=0) + bih_ref[...]
        h = h_ref[...]
        gh = jnp.dot(h.astype(jnp.bfloat16), whh_ref[...],
                     preferred_element_type=jnp.float32) + bhh_ref[...]
        r = jax.nn.sigmoid(gi[:, :H] + gh[:, :H])
        z = jax.nn.sigmoid(gi[:, H:2 * H] + gh[:, H:2 * H])
        n = jnp.tanh(gi[:, 2 * H:] + r * gh[:, 2 * H:])
        h_ref[...] = (1.0 - z) * n + z * h
        return carry

    lax.fori_loop(0, T, step, 0)
    h = jnp.maximum(h_ref[...], 0.0)
    t1 = jnp.dot(h.astype(jnp.bfloat16), wh1_ref[...],
                 preferred_element_type=jnp.float32) + bh1_ref[...]
    t1 = jnp.maximum(t1, 0.0)
    q = jnp.dot(t1.astype(jnp.bfloat16), wh2_ref[...],
                preferred_element_type=jnp.float32) + bh2_ref[...]
    nrm = jnp.sqrt(jnp.sum(q * q, axis=1, keepdims=True))
    q = q / jnp.maximum(nrm, 1e-12)
    q_ref[...] = q.reshape(4, B, CDIM)


def _gru_head(gi, g0, mc, mx, whh_t, bih, bhh, wh1_t, bh1, wh2_t, bh2,
              interpret=False):
    return pl.pallas_call(
        _gru_body,
        out_shape=jax.ShapeDtypeStruct((4, B, CDIM), jnp.float32),
        scratch_shapes=[pltpu.VMEM((4 * B, H), jnp.float32)],
        interpret=interpret,
    )(gi, g0, mc, mx, whh_t, bih, bhh, wh1_t, bh1, wh2_t, bh2)


def _queue_body(q_hbm, k_hbm, lbl_hbm, ptr_hbm, outq_hbm, outp_hbm,
                lbl_v, ptr_v, k_v, np_v, buf_a, buf_b,
                sem_ra, sem_rb, sem_wa, sem_wb):
    cid = lax.axis_index("c")
    sid = lax.axis_index("s")
    wid = cid * _NSUB + sid

    pltpu.sync_copy(lbl_hbm, lbl_v)
    pltpu.sync_copy(ptr_hbm, ptr_v)
    pltpu.sync_copy(k_hbm, k_v)
    iot = lax.iota(jnp.int32, 16)
    lblv = lbl_v[...]
    offs = jnp.zeros((16,), jnp.int32)
    for j in range(16):
        lj = jnp.full((16,), jnp.sum(jnp.where(iot == j, lblv, 0)))
        offs = offs + ((lj == lblv) & (iot > j)).astype(jnp.int32)
    ptrg = plsc.load_gather(ptr_v, [lblv])
    posv = lax.rem(ptrg + offs, jnp.full((16,), KQ, jnp.int32))
    zt = jnp.sum(jnp.where(iot == 0, lblv, 0)) * 0

    base = wid * _WROWS
    bufs = (buf_a, buf_b)
    rsems = (sem_ra, sem_rb)
    wsems = (sem_wa, sem_wb)

    def patch(p, buf):
        r0 = base + p * _PROWS
        cls_p = lax.div(r0, CDIM)
        off = lax.rem(r0, CDIM)
        m = lblv == cls_p
        for j in range(_PROWS):
            col = jnp.full((16,), off + j)
            vals = plsc.load_gather(k_v, [iot, col])
            rows = jnp.full((16,), j + zt)
            plsc.store_scatter(buf, [rows, posv], vals, mask=m)

    rd = pltpu.async_copy(q_hbm.at[pl.ds(base, _PROWS)], bufs[0], rsems[0])
    wr = None
    for p in range(_NP):
        rd.wait()
        if wr is not None:
            wr.wait()
        if p + 1 < _NP:
            rd = pltpu.async_copy(
                q_hbm.at[pl.ds(base + (p + 1) * _PROWS, _PROWS)],
                bufs[(p + 1) % 2], rsems[(p + 1) % 2])
        patch(p, bufs[p % 2])
        wr = pltpu.async_copy(bufs[p % 2],
                              outq_hbm.at[pl.ds(base + p * _PROWS, _PROWS)],
                              wsems[p % 2])
    wr.wait()

    @pl.when((cid == 0) & (sid == 0))
    def _():
        for half in range(2):
            clsv = half * 16 + iot
            cnt = jnp.zeros((16,), jnp.int32)
            for i in range(16):
                liv = jnp.full((16,), jnp.sum(jnp.where(iot == i, lblv, 0)))
                cnt = cnt + (liv == clsv).astype(jnp.int32)
            np_v[pl.ds(half * 16, 16)] = lax.rem(
                ptr_v[pl.ds(half * 16, 16)] + cnt,
                jnp.full((16,), KQ, jnp.int32))
        pltpu.sync_copy(np_v, outp_hbm)


def _queue_update(q2d, k_cls, lbl16, ptr32, interpret=False):
    mesh = plsc.VectorSubcoreMesh(core_axis_name="c", subcore_axis_name="s",
                                  num_cores=2, num_subcores=_NSUB)
    f = functools.partial(
        pl.kernel,
        out_type=(jax.ShapeDtypeStruct((_QROWS, KQ), jnp.float32),
                  jax.ShapeDtypeStruct((32,), jnp.int32)),
        mesh=mesh,
        scratch_types=[
            pltpu.VMEM((16,), jnp.int32),
            pltpu.VMEM((32,), jnp.int32),
            pltpu.VMEM((16, CDIM), jnp.float32),
            pltpu.VMEM((32,), jnp.int32),
            pltpu.VMEM((_PROWS, KQ), jnp.float32),
            pltpu.VMEM((_PROWS, KQ), jnp.float32),
            pltpu.SemaphoreType.DMA,
            pltpu.SemaphoreType.DMA,
            pltpu.SemaphoreType.DMA,
            pltpu.SemaphoreType.DMA,
        ],
        compiler_params=pltpu.CompilerParams(needs_layout_passes=False),
        interpret=interpret,
    )(_queue_body)
    return f(q2d, k_cls, lbl16, ptr32)


def kernel(rgb_anchor, flow_anchor, rgb_shuff, flow_shuff, labels,
           labels_per_frame, W1, b1, g1, be1, W_ih, W_hh, b_ih, b_hh,
           Wh1, bh1, Wh2, bh2, queues, queue_ptrs):
    rk = jax.random.key(42)
    rand = jax.random.uniform(rk, (B, T - 1, 1))
    mask_random = jnp.concatenate(
        [(rand > 0.0).astype(jnp.float32), jnp.ones((B, 1, 1), jnp.float32)],
        axis=1)
    is_bg = (labels_per_frame == 0)[..., None].astype(jnp.float32)
    mask_core_sem = 1.0 - is_bg
    is_bg_sample = (labels == 0).reshape(B, 1, 1)
    mask_core = jnp.where(is_bg_sample, mask_random, mask_core_sem)
    has_action = jnp.sum(mask_core, axis=1, keepdims=True) > 0
    mask_core = jnp.where(has_action, mask_core, mask_random)
    mask_ctx = jnp.where(is_bg_sample, jnp.zeros_like(is_bg), is_bg)
    mc_t = jnp.transpose(mask_core, (1, 0, 2))
    mx_t = jnp.transpose(mask_ctx, (1, 0, 2))

    bf = jnp.bfloat16
    w1r_t = W1[:, :DR].T.astype(bf)
    w1f_t = W1[:, DR:].T.astype(bf)
    wih_t = W_ih.T.astype(bf)
    whh_t = W_hh.T.astype(bf)
    wh1_t = Wh1.T.astype(bf)
    wh2_t = Wh2.T.astype(bf)
    b1r = b1.reshape(1, E)
    g1r = g1.reshape(1, E)
    be1r = be1.reshape(1, E)
    bihr = b_ih.reshape(1, 3 * H)
    bhhr = b_hh.reshape(1, 3 * H)
    bh1r = bh1.reshape(1, H)
    bh2r = bh2.reshape(1, CDIM)

    q = _fused_call(rgb_anchor, flow_anchor, rgb_shuff, flow_shuff,
                    mc_t, mx_t, w1r_t, w1f_t, b1r, g1r, be1r, wih_t,
                    whh_t, bihr, bhhr, wh1_t, bh1r, wh2_t, bh2r)
    q_cls, q_ctx, k_cls, q_shf = q[0], q[1], q[2], q[3]

    lbl16 = labels.astype(jnp.int32)
    ptr32 = jnp.pad(queue_ptrs.astype(jnp.int32), (0, 32 - NC))
    outq, outp = _queue_update(queues.reshape(_QROWS, KQ), k_cls, lbl16, ptr32)
    new_queues = outq.reshape(NC, CDIM, KQ)
    new_ptrs = outp[:NC]

    return (q_cls, k_cls, q_shf, q_ctx, new_queues, new_ptrs)
